# baseline (speedup 1.0000x reference)
import functools

import jax
import jax.numpy as jnp
from jax import lax
from jax.experimental import pallas as pl
from jax.experimental.pallas import tpu as pltpu
from jax.experimental.pallas import tpu_sc as plsc

N = 1_000_000
PAD_N = 1 << 20
BATCH = 16384
HIST = 200
B = BATCH * HIST
NC = 2
NS = 16
NW = NC * NS
PER_W = B // NW
CHUNK = 12_800
CHUNKS = PER_W // CHUNK
UNROLL = 8

TILE_STAGE = PAD_N // NS

_MESH = plsc.VectorSubcoreMesh(core_axis_name="c", subcore_axis_name="s")


@functools.partial(
    pl.kernel,
    out_type=jax.ShapeDtypeStruct((B,), jnp.float32),
    mesh=_MESH,
    compiler_params=pltpu.CompilerParams(use_tc_tiling_on_sc=True),
    scratch_types=[
        pltpu.VMEM_SHARED((PAD_N,), jnp.float32),
        pltpu.VMEM((CHUNK,), jnp.int32),
        pltpu.VMEM((CHUNK,), jnp.int32),
        pltpu.VMEM((CHUNK,), jnp.float32),
        pltpu.VMEM((CHUNK,), jnp.float32),
        pltpu.SemaphoreType.DMA,
        pltpu.SemaphoreType.DMA,
        pltpu.SemaphoreType.DMA,
        pltpu.SemaphoreType.DMA,
        pltpu.SemaphoreType.DMA,
        pltpu.SemaphoreType.DMA,
        pltpu.SemaphoreType.DMA,
    ],
)
def _gather_sc(alpha_hbm, idx_hbm, out_hbm, tbl_sp,
               idx_v0, idx_v1, rows_v0, rows_v1,
               sem_st, sem_i0, sem_i1, sem_g0, sem_g1, sem_o0, sem_o1):
    sem_g = (sem_g0, sem_g1)
    c = lax.axis_index("c")
    s = lax.axis_index("s")
    wid = s * NC + c
    idx_bufs = (idx_v0, idx_v1)
    row_bufs = (rows_v0, rows_v1)
    sem_i = (sem_i0, sem_i1)
    sem_o = (sem_o0, sem_o1)

    def src(j):
        return pl.multiple_of(wid * PER_W + j * CHUNK, 8)

    def idx_load(j):
        b = j & 1
        return pltpu.async_copy(idx_hbm.at[pl.ds(src(j), CHUNK)],
                                idx_bufs[b], sem_i[b])

    def sigmoid_pass(buf):
        def it(i, _):
            base = i * (16 * UNROLL)
            for u in range(UNROLL):
                x = buf[pl.ds(base + u * 16, 16)]
                buf[pl.ds(base + u * 16, 16)] = 1.0 / (1.0 + jnp.exp(-x))
            return 0
        lax.fori_loop(0, CHUNK // (16 * UNROLL), it, 0)

    toff = pl.multiple_of(s * TILE_STAGE, 8)
    h_st = pltpu.async_copy(alpha_hbm.at[pl.ds(toff, TILE_STAGE)],
                            tbl_sp.at[pl.ds(toff, TILE_STAGE)], sem_st)
    h_idx = [None] * CHUNKS
    h_idx[0] = idx_load(0)
    if CHUNKS > 1:
        h_idx[1] = idx_load(1)
    h_st.wait()
    plsc.subcore_barrier()

    h_g = [None] * CHUNKS
    h_out = [None] * CHUNKS
    h_idx[0].wait()
    h_g[0] = pltpu.async_copy(tbl_sp.at[idx_bufs[0]], row_bufs[0], sem_g[0])
    for j in range(CHUNKS):
        b = j & 1
        nb = 1 - b
        h_g[j].wait()
        if j + 2 < CHUNKS:
            h_idx[j + 2] = idx_load(j + 2)
        if j + 1 < CHUNKS:
            if j >= 1:
                h_out[j - 1].wait()
            h_idx[j + 1].wait()
            h_g[j + 1] = pltpu.async_copy(tbl_sp.at[idx_bufs[nb]],
                                          row_bufs[nb], sem_g[nb])
        sigmoid_pass(row_bufs[b])
        h_out[j] = pltpu.async_copy(row_bufs[b],
                                    out_hbm.at[pl.ds(src(j), CHUNK)], sem_o[b])
    h_out[CHUNKS - 2].wait()
    h_out[CHUNKS - 1].wait()


def kernel(idx, alpha):
    alpha_p = jnp.pad(alpha, (0, PAD_N - N))
    flat = idx.reshape(-1).astype(jnp.int32)
    out = _gather_sc(alpha_p, flat)
    return out.reshape(idx.shape)
